# baseline (speedup 1.0000x reference)
import functools

import jax
import jax.numpy as jnp
from jax import lax
from jax.experimental import pallas as pl
from jax.experimental.pallas import tpu as pltpu
from jax.experimental.pallas import tpu_sc as plsc

_DT = 0.001
_N = 100000
_D = 128


def _interp_body(t_hbm, table_hbm, out_hbm, t_s, row_a, row_b, out_s, sem):
    pltpu.sync_copy(t_hbm, t_s)
    t = t_s[0]
    x = t * jnp.float32(1.0 / _DT)
    trunc = x.astype(jnp.int32)
    last_id = jnp.where(x < trunc.astype(jnp.float32), trunc - 1, trunc)
    w = (last_id + 1).astype(jnp.float32) - x
    last_c = jnp.clip(last_id, 0, _N - 1)
    next_c = jnp.clip(last_id + 1, 0, _N - 1)
    cp_a = pltpu.async_copy(table_hbm.at[pl.ds(last_c, 1)], row_a, sem)
    cp_b = pltpu.async_copy(table_hbm.at[pl.ds(next_c, 1)], row_b, sem)
    cp_a.wait()
    cp_b.wait()
    for i in range(_D):
        out_s[i] = w * row_a[0, i] + (jnp.float32(1.0) - w) * row_b[0, i]
    pltpu.sync_copy(out_s, out_hbm)


_interp = functools.partial(
    pl.kernel,
    out_type=jax.ShapeDtypeStruct((_D,), jnp.float32),
    mesh=plsc.ScalarSubcoreMesh(axis_name="c", num_cores=1),
    scratch_types=[
        pltpu.SMEM((1,), jnp.float32),
        pltpu.SMEM((1, _D), jnp.float32),
        pltpu.SMEM((1, _D), jnp.float32),
        pltpu.SMEM((_D,), jnp.float32),
        pltpu.SemaphoreType.DMA,
    ],
)(_interp_body)


def kernel(t, excitation_data):
    return _interp(t.reshape(1), excitation_data)


# speedup vs baseline: 9.7590x; 9.7590x over previous
import jax
import jax.numpy as jnp
from jax.experimental import pallas as pl
from jax.experimental.pallas import tpu as pltpu

_DT = 0.001
_N = 100000
_D = 128


def _tc_body(t_ref, table_ref, out_ref, rows, sem):
    t = t_ref[0]
    x = t / jnp.float32(_DT)
    last_id = jnp.floor(x).astype(jnp.int32)
    w = (last_id + 1).astype(jnp.float32) - x
    last_c = jnp.clip(last_id, 0, _N - 1)
    next_c = jnp.clip(last_id + 1, 0, _N - 1)
    cp_a = pltpu.make_async_copy(
        table_ref.at[pl.ds(last_c, 1)], rows.at[pl.ds(0, 1)], sem
    )
    cp_b = pltpu.make_async_copy(
        table_ref.at[pl.ds(next_c, 1)], rows.at[pl.ds(1, 1)], sem
    )
    cp_a.start()
    cp_b.start()
    cp_a.wait()
    cp_b.wait()
    out_ref[...] = w * rows[pl.ds(0, 1), :] + (jnp.float32(1.0) - w) * rows[
        pl.ds(1, 1), :
    ]


def kernel(t, excitation_data):
    out = pl.pallas_call(
        _tc_body,
        out_shape=jax.ShapeDtypeStruct((1, _D), jnp.float32),
        in_specs=[
            pl.BlockSpec(memory_space=pltpu.SMEM),
            pl.BlockSpec(memory_space=pltpu.HBM),
        ],
        out_specs=pl.BlockSpec(memory_space=pltpu.VMEM),
        scratch_shapes=[
            pltpu.VMEM((2, _D), jnp.float32),
            pltpu.SemaphoreType.DMA,
        ],
    )(t.reshape(1), excitation_data)
    return out.reshape(_D)
